# Initial kernel scaffold; baseline (speedup 1.0000x reference)
import jax
import jax.numpy as jnp
from jax.experimental import pallas as pl


def kernel(input_sequences, target_sequences, token_embedding_table):
    raise NotImplementedError("write your pallas kernel here")


# speedup vs baseline: 1.7151x; 1.7151x over previous
import functools

import jax
import jax.numpy as jnp
from jax import lax
from jax.experimental import pallas as pl
from jax.experimental.pallas import tpu as pltpu
from jax.experimental.pallas import tpu_sc as plsc

_NC = 2
_NS = 16
_L = 16
_NW = _NC * _NS

_V = 1000
_C = 1000
_N = 51200
_RPW = _N // _NW
_CHUNK = 40
_NCHUNK = _RPW // _CHUNK
_SUB = (_CHUNK + _L - 1) // _L


def _lse_body(tab_ref, lse_ref):
    x = tab_ref[...]
    m = jnp.max(x, axis=1)
    s = jnp.sum(jnp.exp(x - m[:, None]), axis=1)
    lse_ref[...] = m + jnp.log(s)


def _loss_body(part_ref, loss_ref):
    loss_ref[...] = jnp.sum(part_ref[...], axis=(0, 1), keepdims=True) * (1.0 / _N)


def _sc_body(table_hbm, inp_hbm, tgt_hbm, lse_hbm,
             out_hbm, part_hbm,
             idx_v, tgt_v, lse_v, rows_v, acc_v,
             gsem0, gsem1, ssem0, ssem1):
    wid = lax.axis_index("s") * _NC + lax.axis_index("c")
    base = wid * _RPW

    pltpu.sync_copy(inp_hbm.at[pl.ds(base, _RPW)], idx_v)
    pltpu.sync_copy(tgt_hbm.at[pl.ds(base, _RPW)], tgt_v)
    pltpu.sync_copy(lse_hbm, lse_v)
    acc_v[...] = jnp.zeros((_L,), jnp.float32)

    gsems = (gsem0, gsem1)
    ssems = (ssem0, ssem1)

    def gather_start(g, b):
        pltpu.async_copy(
            table_hbm.at[idx_v.at[pl.ds(g * _CHUNK, _CHUNK)]],
            rows_v.at[b], gsems[b])

    def gather_wait(b):
        pltpu.make_async_copy(
            table_hbm.at[idx_v.at[pl.ds(0, _CHUNK)]],
            rows_v.at[b], gsems[b]).wait()

    def scatter_start(g, b):
        pltpu.async_copy(
            rows_v.at[b], out_hbm.at[pl.ds(base + g * _CHUNK, _CHUNK)],
            ssems[b])

    def scatter_wait(b):
        pltpu.make_async_copy(
            rows_v.at[b], out_hbm.at[pl.ds(0, _CHUNK)], ssems[b]).wait()

    gather_start(0, 0)
    gather_start(1, 1)

    lane = lax.iota(jnp.int32, 16)

    def chunk_loss(g, b):
        bvec = jnp.full((_L,), b, jnp.int32)
        for j4 in range(_SUB):
            jrow = j4 * _L + lane
            valid = jrow < _CHUNK
            jrow_c = jnp.minimum(jrow, _CHUNK - 1)
            pos = g * _CHUNK + jrow_c
            ivec = plsc.load_gather(idx_v, [pos])
            tvec = plsc.load_gather(tgt_v, [pos])
            tl = plsc.load_gather(rows_v, [bvec, jrow_c, tvec])
            ls = plsc.load_gather(lse_v, [ivec])
            acc_v[...] = acc_v[...] + jnp.where(valid, ls - tl, 0.0)

    def outer(k, carry):
        for b in range(2):
            g = k * 2 + b
            gather_wait(b)
            scatter_start(g, b)
            chunk_loss(g, b)
            scatter_wait(b)

            @pl.when(k < _NCHUNK // 2 - 1)
            def _():
                gather_start(g + 2, b)
        return carry

    lax.fori_loop(0, _NCHUNK // 2, outer, 0)

    pltpu.sync_copy(acc_v, part_hbm.at[wid])


def kernel(input_sequences, target_sequences, token_embedding_table):
    inp = input_sequences.reshape(-1)
    tgt = target_sequences.reshape(-1)

    lse = pl.pallas_call(
        _lse_body,
        out_shape=jax.ShapeDtypeStruct((_V,), jnp.float32),
    )(token_embedding_table)
    lse_pad = jnp.pad(lse, (0, 1024 - _V))

    mesh = plsc.VectorSubcoreMesh(
        core_axis_name="c", subcore_axis_name="s",
        num_cores=_NC, num_subcores=_NS)
    logits_flat, partials = pl.kernel(
        _sc_body,
        out_type=[
            jax.ShapeDtypeStruct((_N, _C), jnp.float32),
            jax.ShapeDtypeStruct((_NW, _L), jnp.float32),
        ],
        mesh=mesh,
        compiler_params=pltpu.CompilerParams(
            needs_layout_passes=False, use_tc_tiling_on_sc=False),
        scratch_types=[
            pltpu.VMEM((_RPW,), jnp.int32),
            pltpu.VMEM((_RPW,), jnp.int32),
            pltpu.VMEM((1024,), jnp.float32),
            pltpu.VMEM((2, _CHUNK, _C), jnp.float32),
            pltpu.VMEM((_L,), jnp.float32),
            pltpu.SemaphoreType.DMA,
            pltpu.SemaphoreType.DMA,
            pltpu.SemaphoreType.DMA,
            pltpu.SemaphoreType.DMA,
        ],
    )(token_embedding_table, inp, tgt, lse_pad)

    loss2d = pl.pallas_call(
        _loss_body,
        out_shape=jax.ShapeDtypeStruct((1, 1), jnp.float32),
    )(partials)
    return logits_flat, loss2d[0, 0]


# speedup vs baseline: 2.3836x; 1.3898x over previous
import jax
import jax.numpy as jnp
from jax import lax
from jax.experimental import pallas as pl
from jax.experimental.pallas import tpu as pltpu
from jax.experimental.pallas import tpu_sc as plsc

_NC = 2
_NS = 16
_L = 16
_NW = _NC * _NS

_V = 1000
_C = 1000
_CP = 1024
_CM = 896
_N = 51200
_RPW = _N // _NW
_CHUNK = 40
_NCHUNK = _RPW // _CHUNK
_EG = 80
_NEG = _RPW // _EG


def _lse_body(tab_ref, lse_ref):
    x = tab_ref[...]
    m = jnp.max(x, axis=1)
    s = jnp.sum(jnp.exp(x - m[:, None]), axis=1)
    lse_ref[...] = m + jnp.log(s)


def _loss_body(part_ref, loss_ref):
    loss_ref[...] = jnp.sum(part_ref[...], axis=(0, 1), keepdims=True) * (1.0 / _N)


def _merge_body(big_ref, tail_ref, out_ref):
    out_ref[...] = tail_ref[...]
    del big_ref


def _gather_body(table_hbm, inp_hbm, out_hbm, tail_hbm,
                 idx_v, rows_v, gsem0, gsem1, ssem0, ssem1):
    wid = lax.axis_index("s") * _NC + lax.axis_index("c")
    base = wid * _RPW

    pltpu.sync_copy(inp_hbm.at[pl.ds(base, _RPW)], idx_v)

    gsems = (gsem0, gsem1)
    ssems = (ssem0, ssem1)

    def gather_start(g, b):
        pltpu.async_copy(
            table_hbm.at[idx_v.at[pl.ds(g * _CHUNK, _CHUNK)]],
            rows_v.at[b], gsems[b])

    def gather_wait(b):
        pltpu.make_async_copy(
            table_hbm.at[idx_v.at[pl.ds(0, _CHUNK)]],
            rows_v.at[b], gsems[b]).wait()

    def scatter_start(g, b):
        r0 = base + g * _CHUNK
        pltpu.async_copy(
            rows_v.at[b, :, pl.ds(0, _CM)],
            out_hbm.at[pl.ds(r0, _CHUNK), pl.ds(0, _CM)],
            ssems[b])
        pltpu.async_copy(
            rows_v.at[b, :, pl.ds(_CM, 128)],
            tail_hbm.at[pl.ds(r0, _CHUNK)],
            ssems[b])

    def scatter_wait(b):
        pltpu.make_async_copy(
            rows_v.at[b, :, pl.ds(0, _CM)],
            out_hbm.at[pl.ds(0, _CHUNK), pl.ds(0, _CM)],
            ssems[b]).wait()
        pltpu.make_async_copy(
            rows_v.at[b, :, pl.ds(_CM, 128)],
            tail_hbm.at[pl.ds(0, _CHUNK)],
            ssems[b]).wait()

    gather_start(0, 0)
    gather_start(1, 1)

    def outer(k, carry):
        for b in range(2):
            g = k * 2 + b
            gather_wait(b)
            scatter_start(g, b)
            scatter_wait(b)

            @pl.when(k < _NCHUNK // 2 - 1)
            def _():
                gather_start(g + 2, b)
        return carry

    lax.fori_loop(0, _NCHUNK // 2, outer, 0)


def _lpart_body(tflat_hbm, inp_hbm, tgt_hbm, lse_hbm,
                part_hbm,
                idx_v, tgt_v, pr_v, tl_v, lse_v, acc_v, gsem):
    wid = lax.axis_index("s") * _NC + lax.axis_index("c")
    base = wid * _RPW

    pltpu.sync_copy(inp_hbm.at[pl.ds(base, _RPW)], idx_v)
    pltpu.sync_copy(tgt_hbm.at[pl.ds(base, _RPW)], tgt_v)
    pltpu.sync_copy(lse_hbm, lse_v)

    def mk_pairs(i, carry):
        sl = pl.ds(i * _L, _L)
        pr_v[sl] = idx_v[sl] * _V + tgt_v[sl]
        return carry
    lax.fori_loop(0, _RPW // _L, mk_pairs, 0)

    def fire(g, carry):
        pltpu.async_copy(
            tflat_hbm.at[pr_v.at[pl.ds(g * _EG, _EG)]],
            tl_v.at[pl.ds(g * _EG, _EG)], gsem)
        return carry
    lax.fori_loop(0, _NEG, fire, 0)

    def drain(g, carry):
        pltpu.make_async_copy(
            tflat_hbm.at[pr_v.at[pl.ds(0, _EG)]],
            tl_v.at[pl.ds(g * _EG, _EG)], gsem).wait()
        return carry
    lax.fori_loop(0, _NEG, drain, 0)

    acc_v[...] = jnp.zeros((_L,), jnp.float32)

    def accum(i, carry):
        sl = pl.ds(i * _L, _L)
        ivec = idx_v[sl]
        ls = plsc.load_gather(lse_v, [ivec])
        acc_v[...] = acc_v[...] + (ls - tl_v[sl])
        return carry
    lax.fori_loop(0, _RPW // _L, accum, 0)

    pltpu.sync_copy(acc_v, part_hbm.at[wid])


def kernel(input_sequences, target_sequences, token_embedding_table):
    inp = input_sequences.reshape(-1)
    tgt = target_sequences.reshape(-1)

    lse = pl.pallas_call(
        _lse_body,
        out_shape=jax.ShapeDtypeStruct((_V,), jnp.float32),
    )(token_embedding_table)
    lse_pad = jnp.pad(lse, (0, 1024 - _V))

    table_pad = jnp.pad(token_embedding_table, ((0, 0), (0, _CP - _C)))
    tflat = token_embedding_table.reshape(-1)

    mesh = plsc.VectorSubcoreMesh(
        core_axis_name="c", subcore_axis_name="s",
        num_cores=_NC, num_subcores=_NS)

    big, tail = pl.kernel(
        _gather_body,
        out_type=[
            jax.ShapeDtypeStruct((_N, _C), jnp.float32),
            jax.ShapeDtypeStruct((_N, 128), jnp.float32),
        ],
        mesh=mesh,
        compiler_params=pltpu.CompilerParams(
            needs_layout_passes=False, use_tc_tiling_on_sc=True),
        scratch_types=[
            pltpu.VMEM((_RPW,), jnp.int32),
            pltpu.VMEM((2, _CHUNK, _CP), jnp.float32),
            pltpu.SemaphoreType.DMA,
            pltpu.SemaphoreType.DMA,
            pltpu.SemaphoreType.DMA,
            pltpu.SemaphoreType.DMA,
        ],
    )(table_pad, inp)

    partials = pl.kernel(
        _lpart_body,
        out_type=jax.ShapeDtypeStruct((_NW, _L), jnp.float32),
        mesh=mesh,
        compiler_params=pltpu.CompilerParams(
            needs_layout_passes=False, use_tc_tiling_on_sc=False),
        scratch_types=[
            pltpu.VMEM((_RPW,), jnp.int32),
            pltpu.VMEM((_RPW,), jnp.int32),
            pltpu.VMEM((_RPW,), jnp.int32),
            pltpu.VMEM((_RPW,), jnp.float32),
            pltpu.VMEM((1024,), jnp.float32),
            pltpu.VMEM((_L,), jnp.float32),
            pltpu.SemaphoreType.DMA,
        ],
    )(tflat, inp, tgt, lse_pad)

    logits_flat = pl.pallas_call(
        _merge_body,
        grid=(8,),
        in_specs=[
            pl.BlockSpec(memory_space=pltpu.MemorySpace.HBM),
            pl.BlockSpec((_N // 8, 128), lambda j: (j, 0)),
        ],
        out_specs=pl.BlockSpec((_N // 8, 128), lambda j: (j, 7)),
        out_shape=jax.ShapeDtypeStruct((_N, _C), jnp.float32),
        input_output_aliases={0: 0},
    )(big, tail)

    loss2d = pl.pallas_call(
        _loss_body,
        out_shape=jax.ShapeDtypeStruct((1, 1), jnp.float32),
    )(partials)
    return logits_flat, loss2d[0, 0]


# speedup vs baseline: 4.3529x; 1.8262x over previous
import jax
import jax.numpy as jnp
from jax import lax
from jax.experimental import pallas as pl
from jax.experimental.pallas import tpu as pltpu
from jax.experimental.pallas import tpu_sc as plsc
from jax.experimental.layout import Layout, with_layout_constraint

_NC = 2
_NS = 16
_L = 16
_NW = _NC * _NS

_V = 1000
_C = 1000
_CP = 1024
_CM = 896
_N = 51200
_RPW = _N // _NW
_CHUNK = 40
_NCHUNK = _RPW // _CHUNK
_EG = 80
_NEG = _RPW // _EG


def _lse_body(tab_ref, lse_ref):
    x = tab_ref[...]
    m = jnp.max(x, axis=1)
    s = jnp.sum(jnp.exp(x - m[:, None]), axis=1)
    lse_ref[...] = m + jnp.log(s)


def _loss_body(part_ref, loss_ref):
    loss_ref[...] = jnp.sum(part_ref[...], axis=(0, 1), keepdims=True) * (1.0 / _N)


def _merge_body(big_ref, tail_ref, out_ref):
    out_ref[...] = tail_ref[...]
    del big_ref


def _gather_body(table_hbm, inp_hbm, out_hbm, tail_hbm,
                 idx_v, rows_v, gsem0, gsem1, ssem0, ssem1):
    wid = lax.axis_index("s") * _NC + lax.axis_index("c")
    base = wid * _RPW

    pltpu.sync_copy(inp_hbm.at[pl.ds(base, _RPW)], idx_v)

    gsems = (gsem0, gsem1)
    ssems = (ssem0, ssem1)

    def gather_start(g, b):
        pltpu.async_copy(
            table_hbm.at[idx_v.at[pl.ds(g * _CHUNK, _CHUNK)]],
            rows_v.at[b], gsems[b])

    def gather_wait(b):
        pltpu.make_async_copy(
            table_hbm.at[idx_v.at[pl.ds(0, _CHUNK)]],
            rows_v.at[b], gsems[b]).wait()

    def scatter_start(g, b):
        r0 = base + g * _CHUNK
        pltpu.async_copy(
            rows_v.at[b, :, pl.ds(0, _CM)],
            out_hbm.at[pl.ds(r0, _CHUNK), pl.ds(0, _CM)],
            ssems[b])
        pltpu.async_copy(
            rows_v.at[b, :, pl.ds(_CM, 128)],
            tail_hbm.at[pl.ds(r0, _CHUNK)],
            ssems[b])

    def scatter_wait(b):
        pltpu.make_async_copy(
            rows_v.at[b, :, pl.ds(0, _CM)],
            out_hbm.at[pl.ds(0, _CHUNK), pl.ds(0, _CM)],
            ssems[b]).wait()
        pltpu.make_async_copy(
            rows_v.at[b, :, pl.ds(_CM, 128)],
            tail_hbm.at[pl.ds(0, _CHUNK)],
            ssems[b]).wait()

    gather_start(0, 0)
    gather_start(1, 1)

    def outer(k, carry):
        for b in range(2):
            g = k * 2 + b
            gather_wait(b)
            scatter_start(g, b)
            scatter_wait(b)

            @pl.when(k < _NCHUNK // 2 - 1)
            def _():
                gather_start(g + 2, b)
        return carry

    lax.fori_loop(0, _NCHUNK // 2, outer, 0)


def _lpart_body(tflat_hbm, inp_hbm, tgt_hbm, lse_hbm,
                part_hbm,
                idx_v, tgt_v, pr_v, tl_v, lse_v, acc_v, gsem):
    wid = lax.axis_index("s") * _NC + lax.axis_index("c")
    base = wid * _RPW

    pltpu.sync_copy(inp_hbm.at[pl.ds(base, _RPW)], idx_v)
    pltpu.sync_copy(tgt_hbm.at[pl.ds(base, _RPW)], tgt_v)
    pltpu.sync_copy(lse_hbm, lse_v)

    def mk_pairs(i, carry):
        sl = pl.ds(i * _L, _L)
        pr_v[sl] = idx_v[sl] * _V + tgt_v[sl]
        return carry
    lax.fori_loop(0, _RPW // _L, mk_pairs, 0)

    def fire(g, carry):
        pltpu.async_copy(
            tflat_hbm.at[pr_v.at[pl.ds(g * _EG, _EG)]],
            tl_v.at[pl.ds(g * _EG, _EG)], gsem)
        return carry
    lax.fori_loop(0, _NEG, fire, 0)

    def drain(g, carry):
        pltpu.make_async_copy(
            tflat_hbm.at[pr_v.at[pl.ds(0, _EG)]],
            tl_v.at[pl.ds(g * _EG, _EG)], gsem).wait()
        return carry
    lax.fori_loop(0, _NEG, drain, 0)

    acc_v[...] = jnp.zeros((_L,), jnp.float32)

    def accum(i, carry):
        sl = pl.ds(i * _L, _L)
        ivec = idx_v[sl]
        ls = plsc.load_gather(lse_v, [ivec])
        acc_v[...] = acc_v[...] + (ls - tl_v[sl])
        return carry
    lax.fori_loop(0, _RPW // _L, accum, 0)

    pltpu.sync_copy(acc_v, part_hbm.at[wid])


def kernel(input_sequences, target_sequences, token_embedding_table):
    inp = input_sequences.reshape(-1)
    tgt = target_sequences.reshape(-1)

    lse = pl.pallas_call(
        _lse_body,
        out_shape=jax.ShapeDtypeStruct((_V,), jnp.float32),
    )(token_embedding_table)
    lse_pad = jnp.pad(lse, (0, 1024 - _V))

    table_pad = jnp.pad(token_embedding_table, ((0, 0), (0, _CP - _C)))
    tflat = token_embedding_table.reshape(-1)

    mesh = plsc.VectorSubcoreMesh(
        core_axis_name="c", subcore_axis_name="s",
        num_cores=_NC, num_subcores=_NS)

    big, tail = pl.kernel(
        _gather_body,
        out_type=[
            jax.ShapeDtypeStruct((_N, _C), jnp.float32),
            jax.ShapeDtypeStruct((_N, 128), jnp.float32),
        ],
        mesh=mesh,
        compiler_params=pltpu.CompilerParams(
            needs_layout_passes=False, use_tc_tiling_on_sc=True),
        scratch_types=[
            pltpu.VMEM((_RPW,), jnp.int32),
            pltpu.VMEM((2, _CHUNK, _CP), jnp.float32),
            pltpu.SemaphoreType.DMA,
            pltpu.SemaphoreType.DMA,
            pltpu.SemaphoreType.DMA,
            pltpu.SemaphoreType.DMA,
        ],
    )(table_pad, inp)

    partials = pl.kernel(
        _lpart_body,
        out_type=jax.ShapeDtypeStruct((_NW, _L), jnp.float32),
        mesh=mesh,
        compiler_params=pltpu.CompilerParams(
            needs_layout_passes=False, use_tc_tiling_on_sc=False),
        scratch_types=[
            pltpu.VMEM((_RPW,), jnp.int32),
            pltpu.VMEM((_RPW,), jnp.int32),
            pltpu.VMEM((_RPW,), jnp.int32),
            pltpu.VMEM((_RPW,), jnp.float32),
            pltpu.VMEM((1024,), jnp.float32),
            pltpu.VMEM((_L,), jnp.float32),
            pltpu.SemaphoreType.DMA,
        ],
    )(tflat, inp, tgt, lse_pad)

    logits_flat = pl.pallas_call(
        _merge_body,
        grid=(8,),
        in_specs=[
            pl.BlockSpec(memory_space=pltpu.MemorySpace.HBM),
            pl.BlockSpec((_N // 8, 128), lambda j: (j, 0)),
        ],
        out_specs=pl.BlockSpec((_N // 8, 128), lambda j: (j, 7)),
        out_shape=jax.ShapeDtypeStruct((_N, _C), jnp.float32),
        input_output_aliases={0: 0},
    )(big, tail)

    logits_flat = with_layout_constraint(
        logits_flat, Layout(major_to_minor=(0, 1)))

    loss2d = pl.pallas_call(
        _loss_body,
        out_shape=jax.ShapeDtypeStruct((1, 1), jnp.float32),
    )(partials)
    return logits_flat, loss2d[0, 0]
